# baseline (speedup 1.0000x reference)
import functools

import jax
import jax.numpy as jnp
from jax import lax
from jax.experimental import pallas as pl
from jax.experimental.pallas import tpu as pltpu
from jax.experimental.pallas import tpu_sc as plsc

NC = 2
NS = 16
NW = NC * NS

DEG = 32
D = 128
B_PAD = 10240
B_PER_W = B_PAD // NW
N_SLICE = 10
SLICE_ROWS = B_PER_W // N_SLICE
N_SELF = 4
SELF_ROWS = B_PER_W // N_SELF


def _sc_gather_body(neigh_hbm, nodes_hbm, table_hbm,
                    self_out, nsum_out,
                    idxs, sidx, oslab, sslab,
                    sem0, sem1, sem2, sem3, sem4, sem5, sem6, sem7,
                    sem8, sem9, ssem):
    wid = lax.axis_index("s") * NC + lax.axis_index("c")

    pltpu.sync_copy(nodes_hbm.at[pl.ds(wid * B_PER_W, B_PER_W)], sidx)
    pltpu.sync_copy(neigh_hbm.at[pl.ds(wid * B_PER_W * DEG, B_PER_W * DEG)],
                    idxs)

    for j in range(N_SELF):
        pltpu.async_copy(
            table_hbm.at[sidx.at[pl.ds(j * SELF_ROWS, SELF_ROWS)]],
            sslab.at[pl.ds(j * SELF_ROWS, SELF_ROWS)], ssem)

    sems = (sem0, sem1, sem2, sem3, sem4, sem5, sem6, sem7, sem8, sem9)

    def gather(k, b, add):
        src = table_hbm.at[idxs.at[pl.ds(k * B_PER_W + b * SLICE_ROWS,
                                         SLICE_ROWS)]]
        dst = oslab.at[pl.ds(b * SLICE_ROWS, SLICE_ROWS)]
        pltpu.async_copy(src, dst, sems[b], add=add)

    for b in range(N_SLICE):
        gather(0, b, False)

    def outer(k, carry):
        for b in range(N_SLICE):
            src = table_hbm.at[idxs.at[pl.ds(k * B_PER_W + b * SLICE_ROWS,
                                             SLICE_ROWS)]]
            dst = oslab.at[pl.ds(b * SLICE_ROWS, SLICE_ROWS)]
            pltpu.make_async_copy(src, dst, sems[b]).wait()

            @pl.when(k + 1 < DEG)
            def _(k=k, b=b):
                gather(k + 1, b, True)
        return carry

    lax.fori_loop(0, DEG, outer, 0)

    pltpu.sync_copy(oslab, nsum_out.at[pl.ds(wid * B_PER_W, B_PER_W)])

    for j in range(N_SELF):
        pltpu.make_async_copy(
            table_hbm.at[sidx.at[pl.ds(j * SELF_ROWS, SELF_ROWS)]],
            sslab.at[pl.ds(j * SELF_ROWS, SELF_ROWS)], ssem).wait()
    pltpu.sync_copy(sslab, self_out.at[pl.ds(wid * B_PER_W, B_PER_W)])


@jax.jit
def _sc_gather(neigh_flat, nodes_flat, table):
    mesh = plsc.VectorSubcoreMesh(core_axis_name="c", subcore_axis_name="s",
                                  num_cores=NC, num_subcores=NS)
    fn = functools.partial(
        pl.kernel,
        out_type=(
            jax.ShapeDtypeStruct((B_PAD, D), jnp.float32),
            jax.ShapeDtypeStruct((B_PAD, D), jnp.float32),
        ),
        mesh=mesh,
        scratch_types=[
            pltpu.VMEM((B_PER_W * DEG,), jnp.int32),
            pltpu.VMEM((B_PER_W,), jnp.int32),
            pltpu.VMEM((B_PER_W, D), jnp.float32),
            pltpu.VMEM((B_PER_W, D), jnp.float32),
        ] + [pltpu.SemaphoreType.DMA] * 11,
    )(_sc_gather_body)
    return fn(neigh_flat, nodes_flat, table)


def _mm_body(self_ref, nsum_ref, wa_ref, wb_ref, b_ref, o_ref):
    x = (jnp.dot(self_ref[...], wa_ref[...], preferred_element_type=jnp.float32)
         + jnp.dot(nsum_ref[...], wb_ref[...], preferred_element_type=jnp.float32)
         + b_ref[...])
    o_ref[...] = jnp.maximum(x, 0.0)


def _combine(self_rows, nsum, wa, wb_scaled, b2d):
    blk = 1024
    return pl.pallas_call(
        _mm_body,
        grid=(B_PAD // blk,),
        in_specs=[
            pl.BlockSpec((blk, D), lambda i: (i, 0)),
            pl.BlockSpec((blk, D), lambda i: (i, 0)),
            pl.BlockSpec((D, D), lambda i: (0, 0)),
            pl.BlockSpec((D, D), lambda i: (0, 0)),
            pl.BlockSpec((1, D), lambda i: (0, 0)),
        ],
        out_specs=pl.BlockSpec((blk, D), lambda i: (i, 0)),
        out_shape=jax.ShapeDtypeStruct((B_PAD, D), jnp.float32),
    )(self_rows, nsum, wa, wb_scaled, b2d)


def kernel(nodes, neighbors, table, W1, b1):
    B = nodes.shape[0]
    pad = B_PAD - B
    n_nodes = table.shape[0]
    pad_nodes = (jnp.arange(pad, dtype=jnp.int32) * 131) % n_nodes
    pad_neigh = ((jnp.arange(pad * DEG, dtype=jnp.int32) * 131) % n_nodes
                 ).reshape(pad, DEG)
    nodes_p = jnp.concatenate([nodes, pad_nodes])
    neigh_p = jnp.concatenate([neighbors, pad_neigh], axis=0)
    neigh_flat = neigh_p.reshape(NW, B_PER_W, DEG).transpose(0, 2, 1).reshape(-1)
    self_rows, nsum = _sc_gather(neigh_flat, nodes_p, table)
    wa = W1[:D]
    wb_scaled = W1[D:] * (1.0 / DEG)
    out = _combine(self_rows, nsum, wa, wb_scaled, b1.reshape(1, D))
    return out[:B]


# speedup vs baseline: 1.0692x; 1.0692x over previous
import functools

import jax
import jax.numpy as jnp
from jax import lax
from jax.experimental import pallas as pl
from jax.experimental.pallas import tpu as pltpu
from jax.experimental.pallas import tpu_sc as plsc

NC = 2
NS = 16
NW = NC * NS

DEG = 32
D = 128
B_PAD = 10240
B_PER_W = B_PAD // NW
N_SLICE = 8
SLICE_ROWS = B_PER_W // N_SLICE
N_SELF = 4
SELF_ROWS = B_PER_W // N_SELF


def _sc_gather_body(neigh_hbm, nodes_hbm, table_hbm,
                    self_out, nsum_out,
                    idxs, sidx, oslab, sslab,
                    sem0, sem1, sem2, sem3, sem4, sem5, sem6, sem7,
                    ssem):
    wid = lax.axis_index("s") * NC + lax.axis_index("c")

    pltpu.sync_copy(nodes_hbm.at[pl.ds(wid * B_PER_W, B_PER_W)], sidx)
    pltpu.sync_copy(neigh_hbm.at[pl.ds(wid * B_PER_W * DEG, B_PER_W * DEG)],
                    idxs)

    for j in range(N_SELF):
        pltpu.async_copy(
            table_hbm.at[sidx.at[pl.ds(j * SELF_ROWS, SELF_ROWS)]],
            sslab.at[pl.ds(j * SELF_ROWS, SELF_ROWS)], ssem)

    sems = (sem0, sem1, sem2, sem3, sem4, sem5, sem6, sem7)

    def gather(k, b, add):
        src = table_hbm.at[idxs.at[pl.ds(k * B_PER_W + b * SLICE_ROWS,
                                         SLICE_ROWS)]]
        dst = oslab.at[pl.ds(b * SLICE_ROWS, SLICE_ROWS)]
        pltpu.async_copy(src, dst, sems[b], add=add)

    for b in range(N_SLICE):
        gather(0, b, False)

    def outer(k, carry):
        for b in range(N_SLICE):
            src = table_hbm.at[idxs.at[pl.ds(k * B_PER_W + b * SLICE_ROWS,
                                             SLICE_ROWS)]]
            dst = oslab.at[pl.ds(b * SLICE_ROWS, SLICE_ROWS)]
            pltpu.make_async_copy(src, dst, sems[b]).wait()

            @pl.when(k + 1 < DEG)
            def _(k=k, b=b):
                gather(k + 1, b, True)
        return carry

    lax.fori_loop(0, DEG, outer, 0)

    pltpu.sync_copy(oslab, nsum_out.at[pl.ds(wid * B_PER_W, B_PER_W)])

    for j in range(N_SELF):
        pltpu.make_async_copy(
            table_hbm.at[sidx.at[pl.ds(j * SELF_ROWS, SELF_ROWS)]],
            sslab.at[pl.ds(j * SELF_ROWS, SELF_ROWS)], ssem).wait()
    pltpu.sync_copy(sslab, self_out.at[pl.ds(wid * B_PER_W, B_PER_W)])


@jax.jit
def _sc_gather(neigh_flat, nodes_flat, table):
    mesh = plsc.VectorSubcoreMesh(core_axis_name="c", subcore_axis_name="s",
                                  num_cores=NC, num_subcores=NS)
    fn = functools.partial(
        pl.kernel,
        out_type=(
            jax.ShapeDtypeStruct((B_PAD, D), jnp.float32),
            jax.ShapeDtypeStruct((B_PAD, D), jnp.float32),
        ),
        mesh=mesh,
        scratch_types=[
            pltpu.VMEM((B_PER_W * DEG,), jnp.int32),
            pltpu.VMEM((B_PER_W,), jnp.int32),
            pltpu.VMEM((B_PER_W, D), jnp.float32),
            pltpu.VMEM((B_PER_W, D), jnp.float32),
        ] + [pltpu.SemaphoreType.DMA] * 9,
    )(_sc_gather_body)
    return fn(neigh_flat, nodes_flat, table)


def _mm_body(self_ref, nsum_ref, wa_ref, wb_ref, b_ref, o_ref):
    x = (jnp.dot(self_ref[...], wa_ref[...], preferred_element_type=jnp.float32)
         + jnp.dot(nsum_ref[...], wb_ref[...], preferred_element_type=jnp.float32)
         + b_ref[...])
    o_ref[...] = jnp.maximum(x, 0.0)


def _combine(self_rows, nsum, wa, wb_scaled, b2d, n_out):
    blk = 1024
    return pl.pallas_call(
        _mm_body,
        grid=(B_PAD // blk,),
        in_specs=[
            pl.BlockSpec((blk, D), lambda i: (i, 0)),
            pl.BlockSpec((blk, D), lambda i: (i, 0)),
            pl.BlockSpec((D, D), lambda i: (0, 0)),
            pl.BlockSpec((D, D), lambda i: (0, 0)),
            pl.BlockSpec((1, D), lambda i: (0, 0)),
        ],
        out_specs=pl.BlockSpec((blk, D), lambda i: (i, 0)),
        out_shape=jax.ShapeDtypeStruct((n_out, D), jnp.float32),
    )(self_rows, nsum, wa, wb_scaled, b2d)


def kernel(nodes, neighbors, table, W1, b1):
    B = nodes.shape[0]
    pad = B_PAD - B
    n_nodes = table.shape[0]
    pad_nodes = (jnp.arange(pad, dtype=jnp.int32) * 131) % n_nodes
    pad_neigh = ((jnp.arange(pad * DEG, dtype=jnp.int32) * 131) % n_nodes
                 ).reshape(pad, DEG)
    nodes_p = jnp.concatenate([nodes, pad_nodes])
    neigh_p = jnp.concatenate([neighbors, pad_neigh], axis=0)
    neigh_flat = neigh_p.reshape(NW, B_PER_W, DEG).transpose(0, 2, 1).reshape(-1)
    self_rows, nsum = _sc_gather(neigh_flat, nodes_p, table)
    wa = W1[:D]
    wb_scaled = W1[D:] * (1.0 / DEG)
    return _combine(self_rows, nsum, wa, wb_scaled, b1.reshape(1, D), B)
